# baseline (speedup 1.0000x reference)
import functools

import jax
import jax.numpy as jnp
from jax.experimental import pallas as pl
from jax.experimental.pallas import tpu as pltpu

_VMEM_LIMIT = 48 * 1024 * 1024


def _cparams():
    return pltpu.CompilerParams(
        dimension_semantics=("parallel", "parallel"),
        vmem_limit_bytes=_VMEM_LIMIT,
        internal_scratch_in_bytes=8 * 1024 * 1024)


def _chain_kernel(tP, aP, bP, tG, aG, bG, *rest,
                  th, wp, wdata, hb, couts, pool, sse, keep,
                  wpair, nchw, nb):
    nl = len(couts)
    wrefs = rest[:2 * nl]
    outs = rest[2 * nl:]
    i = pl.program_id(1)
    nr0 = th + 2 * hb

    def build(t_ref, a_ref, bo_ref):
        top = jnp.where(i > 0, t_ref[0], jnp.zeros_like(t_ref[0]))
        bot = jnp.where(i < nb - 1, bo_ref[0], jnp.zeros_like(bo_ref[0]))
        if nchw:
            x3 = jnp.concatenate([top, a_ref[0], bot], axis=1)
            xt = jnp.transpose(x3).astype(jnp.bfloat16)
            x4 = xt.reshape(nr0, wdata, x3.shape[0])
            band = jnp.concatenate(
                [x4, jnp.zeros((nr0, wp - wdata, x3.shape[0]), jnp.bfloat16)],
                axis=1).reshape(nr0 * wp, x3.shape[0])
        else:
            band = jnp.concatenate([top, a_ref[0], bot], axis=0)
            if wpair:
                c = band.shape[-1] // 2
                band = jnp.maximum(band[:, :c], band[:, c:])
        return band

    def stage(bandP, bandG, l):
        nrows = nr0 - 2 * (l + 1)
        m = nrows * wp
        cout = couts[l]

        w_ref = wrefs[2 * l]
        cin = w_ref.shape[0] // 3

        off = hb - (l + 1)
        lo = jnp.where(i > 0, 0, off)
        hi = jnp.where(i < nb - 1, nrows, nrows - off)
        rows = jax.lax.broadcasted_iota(jnp.int32, (nrows, wp, cout), 0)
        cols = jax.lax.broadcasted_iota(jnp.int32, (nrows, wp, cout), 1)
        keepm = (cols < wdata) & (rows >= lo) & (rows < hi)

        def epi(uh):
            o = jnp.maximum(uh, 0.0).astype(jnp.bfloat16).reshape(
                nrows, wp, cout)
            return jnp.where(keepm, o, jnp.zeros_like(o)).reshape(m, cout)

        def combine(z, mh):
            return (pltpu.roll(z[:, :cout], 1, axis=0) + z[:, cout:2 * cout]
                    + pltpu.roll(z[:, 2 * cout:], mh - 1, axis=0)
                    + wrefs[2 * l + 1][...])

        if 3 * cin <= 256:
            def fold(band):
                return jnp.concatenate(
                    [band[0:m], band[wp:wp + m], band[2 * wp:2 * wp + m]],
                    axis=1)
            z2 = jnp.dot(jnp.concatenate([fold(bandP), fold(bandG)], axis=0),
                         w_ref[...], preferred_element_type=jnp.float32)
            u = combine(z2, 2 * m)
            return epi(u[0:m]), epi(u[m:2 * m])

        def one(band):
            z = jnp.dot(band[0:m], w_ref[0:cin],
                        preferred_element_type=jnp.float32)
            z = z + jnp.dot(band[wp:wp + m], w_ref[cin:2 * cin],
                            preferred_element_type=jnp.float32)
            z = z + jnp.dot(band[2 * wp:2 * wp + m], w_ref[2 * cin:3 * cin],
                            preferred_element_type=jnp.float32)
            return epi(combine(z, m))

        return one(bandP), one(bandG)

    def run(t_ref, a_ref, bo_ref):
        return build(t_ref, a_ref, bo_ref)

    bandP = run(tP, aP, bP)
    bandG = run(tG, aG, bG)
    for l in range(nl):
        bandP, bandG = stage(bandP, bandG, l)
    if hb > nl:
        bandP = bandP[(hb - nl) * wp:(hb - nl) * wp + th * wp]
        bandG = bandG[(hb - nl) * wp:(hb - nl) * wp + th * wp]
    oP, oG = bandP, bandG
    cout = couts[-1]
    j = 0
    if keep:
        if pool:
            pPr = oP.reshape(th // 2, 2, wp, cout)
            pGr = oG.reshape(th // 2, 2, wp, cout)
            outs[j][0] = jnp.maximum(pPr[:, 0], pPr[:, 1]).reshape(
                (th // 2) * wp, cout)
            outs[j + 1][0] = jnp.maximum(pGr[:, 0], pGr[:, 1]).reshape(
                (th // 2) * wp, cout)
        else:
            outs[j][0] = oP
            outs[j + 1][0] = oG
        j += 2
    if sse:
        d = oP.astype(jnp.float32) - oG.astype(jnp.float32)
        s1 = jnp.sum((d * d).reshape(th, wp, cout), axis=0)
        outs[j][0] = jnp.sum(s1.reshape(wp // 8, 8, cout), axis=0)


def _conv_chain(xP, xG, wbs, *, h, wp, wdata, th, hb,
                pool=False, sse=False, keep=True, wpair=False, nchw=False):
    n = xP.shape[0]
    couts = [wb[0].shape[0] for wb in wbs]
    th = min(th, h)
    nb = h // th

    if nchw:
        cin0 = xP.shape[1]
        top = pl.BlockSpec((1, cin0, hb * wdata),
                           lambda b, i: (b, 0, jnp.maximum(i * (th // hb) - 1, 0)))
        main = pl.BlockSpec((1, cin0, th * wdata), lambda b, i: (b, 0, i))
        bot = pl.BlockSpec(
            (1, cin0, hb * wdata),
            lambda b, i: (b, 0, jnp.minimum((i + 1) * (th // hb), h // hb - 1)))
    else:
        clanes = xP.shape[2]
        top = pl.BlockSpec((1, hb * wp, clanes),
                           lambda b, i: (b, jnp.maximum(i * (th // hb) - 1, 0), 0))
        main = pl.BlockSpec((1, th * wp, clanes), lambda b, i: (b, i, 0))
        bot = pl.BlockSpec(
            (1, hb * wp, clanes),
            lambda b, i: (b, jnp.minimum((i + 1) * (th // hb), h // hb - 1), 0))

    wargs, wspecs = [], []
    for w_oihw, bias in wbs:
        cin = w_oihw.shape[1]
        cout = w_oihw.shape[0]
        wargs.append(jnp.transpose(w_oihw, (2, 1, 3, 0)).reshape(
            3 * cin, 3 * cout).astype(jnp.bfloat16))
        wspecs.append(pl.BlockSpec((3 * cin, 3 * cout), lambda b, i: (0, 0)))
        wargs.append(bias.reshape(1, cout).astype(jnp.float32))
        wspecs.append(pl.BlockSpec((1, cout), lambda b, i: (0, 0)))

    out_shapes, out_specs = [], []
    cL = couts[-1]
    if keep:
        rows = (th // 2) * wp if pool else th * wp
        hrows = (h // 2) * wp if pool else h * wp
        out_shapes += [jax.ShapeDtypeStruct((n, hrows, cL), jnp.bfloat16)] * 2
        ospec = pl.BlockSpec((1, rows, cL), lambda b, i: (b, i, 0))
        out_specs += [ospec, ospec]
    if sse:
        out_shapes.append(jax.ShapeDtypeStruct((n * nb, 8, cL), jnp.float32))
        out_specs.append(
            pl.BlockSpec((1, 8, cL), lambda b, i: (b * nb + i, 0, 0)))

    body = functools.partial(_chain_kernel, th=th, wp=wp, wdata=wdata, hb=hb,
                             couts=couts, pool=pool, sse=sse, keep=keep,
                             wpair=wpair, nchw=nchw, nb=nb)
    return pl.pallas_call(
        body,
        out_shape=tuple(out_shapes),
        grid_spec=pltpu.PrefetchScalarGridSpec(
            num_scalar_prefetch=0,
            grid=(n, nb),
            in_specs=[top, main, bot, top, main, bot] + wspecs,
            out_specs=tuple(out_specs),
        ),
        compiler_params=_cparams(),
    )(xP, xP, xP, xG, xG, xG, *wargs)


def _pair_view(x, h, wp):
    n, _, c = x.shape
    return x.reshape(n, h, wp // 2, 2 * c).reshape(n, h * (wp // 2), 2 * c)


def kernel(pred_im, gt,
           conv1_1_w, conv1_1_b, conv1_2_w, conv1_2_b,
           conv2_1_w, conv2_1_b, conv2_2_w, conv2_2_b,
           conv3_1_w, conv3_1_b, conv3_2_w, conv3_2_b,
           conv3_3_w, conv3_3_b):
    n, nc, h, w = pred_im.shape
    wp1 = ((w + 2) + 15) // 16 * 16 + 16
    xP = pred_im.reshape(n, nc, h * w)
    xG = gt.reshape(n, nc, h * w)
    wp2, wp3 = wp1 // 2, wp1 // 4

    yP, yG = _conv_chain(
        xP, xG, [(conv1_1_w, conv1_1_b)],
        h=h, wp=wp1, wdata=w, th=16, hb=1, nchw=True)
    pP, pG, s1 = _conv_chain(
        yP, yG, [(conv1_2_w, conv1_2_b)],
        h=h, wp=wp1, wdata=w, th=16, hb=1, pool=True, sse=True)
    yP, yG = _conv_chain(
        _pair_view(pP, h // 2, wp1), _pair_view(pG, h // 2, wp1),
        [(conv2_1_w, conv2_1_b)],
        h=h // 2, wp=wp2, wdata=w // 2, th=32, hb=1, wpair=True)
    pP, pG, s2 = _conv_chain(
        yP, yG, [(conv2_2_w, conv2_2_b)],
        h=h // 2, wp=wp2, wdata=w // 2, th=32, hb=1, pool=True, sse=True)
    yP, yG = _conv_chain(
        _pair_view(pP, h // 4, wp2), _pair_view(pG, h // 4, wp2),
        [(conv3_1_w, conv3_1_b)],
        h=h // 4, wp=wp3, wdata=w // 4, th=32, hb=1, wpair=True)
    yP, yG = _conv_chain(
        yP, yG, [(conv3_2_w, conv3_2_b)],
        h=h // 4, wp=wp3, wdata=w // 4, th=32, hb=1)
    (s3,) = _conv_chain(
        yP, yG, [(conv3_3_w, conv3_3_b)],
        h=h // 4, wp=wp3, wdata=w // 4, th=32, hb=1,
        sse=True, keep=False)

    n1 = n * h * w * conv1_2_w.shape[0]
    n2 = n * (h // 2) * (w // 2) * conv2_2_w.shape[0]
    n3 = n * (h // 4) * (w // 4) * conv3_3_w.shape[0]
    return (jnp.sum(s1) / n1 + jnp.sum(s2) / n2 + jnp.sum(s3) / n3) / 3.0


# speedup vs baseline: 1.0407x; 1.0407x over previous
import functools

import jax
import jax.numpy as jnp
from jax.experimental import pallas as pl
from jax.experimental.pallas import tpu as pltpu

_VMEM_LIMIT = 48 * 1024 * 1024


def _cparams():
    return pltpu.CompilerParams(
        dimension_semantics=("parallel", "parallel"),
        vmem_limit_bytes=_VMEM_LIMIT,
        internal_scratch_in_bytes=8 * 1024 * 1024)


def _chain_kernel(tP, aP, bP, tG, aG, bG, *rest,
                  th, wp, wdata, hb, couts, pool, sse, keep,
                  wpair, nchw, nb):
    nl = len(couts)
    wrefs = rest[:2 * nl]
    outs = rest[2 * nl:]
    i = pl.program_id(1)
    nr0 = th + 2 * hb

    def build(t_ref, a_ref, bo_ref):
        top = jnp.where(i > 0, t_ref[0], jnp.zeros_like(t_ref[0]))
        bot = jnp.where(i < nb - 1, bo_ref[0], jnp.zeros_like(bo_ref[0]))
        if nchw:
            x3 = jnp.concatenate([top, a_ref[0], bot], axis=1)
            xt = jnp.transpose(x3).astype(jnp.bfloat16)
            x4 = xt.reshape(nr0, wdata, x3.shape[0])
            band = jnp.concatenate(
                [x4, jnp.zeros((nr0, wp - wdata, x3.shape[0]), jnp.bfloat16)],
                axis=1).reshape(nr0 * wp, x3.shape[0])
        else:
            band = jnp.concatenate([top, a_ref[0], bot], axis=0)
            if wpair:
                c = band.shape[-1] // 2
                band = jnp.maximum(band[:, :c], band[:, c:])
        return band

    def stage(bandP, bandG, l):
        nrows = nr0 - 2 * (l + 1)
        m = nrows * wp
        cout = couts[l]

        w_ref = wrefs[2 * l]
        cin = w_ref.shape[0] // 3

        off = hb - (l + 1)
        lo = jnp.where(i > 0, 0, off)
        hi = jnp.where(i < nb - 1, nrows, nrows - off)
        rows = jax.lax.broadcasted_iota(jnp.int32, (nrows, wp, cout), 0)
        cols = jax.lax.broadcasted_iota(jnp.int32, (nrows, wp, cout), 1)
        keepm = (cols < wdata) & (rows >= lo) & (rows < hi)

        def epi(uh):
            o = jnp.maximum(uh, 0.0).astype(jnp.bfloat16).reshape(
                nrows, wp, cout)
            return jnp.where(keepm, o, jnp.zeros_like(o)).reshape(m, cout)

        def combine(z, mh):
            return (pltpu.roll(z[:, :cout], 1, axis=0) + z[:, cout:2 * cout]
                    + pltpu.roll(z[:, 2 * cout:], mh - 1, axis=0)
                    + wrefs[2 * l + 1][...])

        def fold(band):
            return jnp.concatenate(
                [band[0:m], band[wp:wp + m], band[2 * wp:2 * wp + m]],
                axis=1)
        z2 = jnp.dot(jnp.concatenate([fold(bandP), fold(bandG)], axis=0),
                     w_ref[...], preferred_element_type=jnp.float32)
        u = combine(z2, 2 * m)
        return epi(u[0:m]), epi(u[m:2 * m])

    def run(t_ref, a_ref, bo_ref):
        return build(t_ref, a_ref, bo_ref)

    bandP = run(tP, aP, bP)
    bandG = run(tG, aG, bG)
    for l in range(nl):
        bandP, bandG = stage(bandP, bandG, l)
    if hb > nl:
        bandP = bandP[(hb - nl) * wp:(hb - nl) * wp + th * wp]
        bandG = bandG[(hb - nl) * wp:(hb - nl) * wp + th * wp]
    oP, oG = bandP, bandG
    cout = couts[-1]
    j = 0
    if keep:
        if pool:
            pPr = oP.reshape(th // 2, 2, wp, cout)
            pGr = oG.reshape(th // 2, 2, wp, cout)
            outs[j][0] = jnp.maximum(pPr[:, 0], pPr[:, 1]).reshape(
                (th // 2) * wp, cout)
            outs[j + 1][0] = jnp.maximum(pGr[:, 0], pGr[:, 1]).reshape(
                (th // 2) * wp, cout)
        else:
            outs[j][0] = oP
            outs[j + 1][0] = oG
        j += 2
    if sse:
        d = oP.astype(jnp.float32) - oG.astype(jnp.float32)
        s1 = jnp.sum((d * d).reshape(th, wp, cout), axis=0)
        outs[j][0] = jnp.sum(s1.reshape(wp // 8, 8, cout), axis=0)


def _conv_chain(xP, xG, wbs, *, h, wp, wdata, th, hb,
                pool=False, sse=False, keep=True, wpair=False, nchw=False):
    n = xP.shape[0]
    couts = [wb[0].shape[0] for wb in wbs]
    th = min(th, h)
    nb = h // th

    if nchw:
        cin0 = xP.shape[1]
        top = pl.BlockSpec((1, cin0, hb * wdata),
                           lambda b, i: (b, 0, jnp.maximum(i * (th // hb) - 1, 0)))
        main = pl.BlockSpec((1, cin0, th * wdata), lambda b, i: (b, 0, i))
        bot = pl.BlockSpec(
            (1, cin0, hb * wdata),
            lambda b, i: (b, 0, jnp.minimum((i + 1) * (th // hb), h // hb - 1)))
    else:
        clanes = xP.shape[2]
        top = pl.BlockSpec((1, hb * wp, clanes),
                           lambda b, i: (b, jnp.maximum(i * (th // hb) - 1, 0), 0))
        main = pl.BlockSpec((1, th * wp, clanes), lambda b, i: (b, i, 0))
        bot = pl.BlockSpec(
            (1, hb * wp, clanes),
            lambda b, i: (b, jnp.minimum((i + 1) * (th // hb), h // hb - 1), 0))

    wargs, wspecs = [], []
    for w_oihw, bias in wbs:
        cin = w_oihw.shape[1]
        cout = w_oihw.shape[0]
        wargs.append(jnp.transpose(w_oihw, (2, 1, 3, 0)).reshape(
            3 * cin, 3 * cout).astype(jnp.bfloat16))
        wspecs.append(pl.BlockSpec((3 * cin, 3 * cout), lambda b, i: (0, 0)))
        wargs.append(bias.reshape(1, cout).astype(jnp.float32))
        wspecs.append(pl.BlockSpec((1, cout), lambda b, i: (0, 0)))

    out_shapes, out_specs = [], []
    cL = couts[-1]
    if keep:
        rows = (th // 2) * wp if pool else th * wp
        hrows = (h // 2) * wp if pool else h * wp
        out_shapes += [jax.ShapeDtypeStruct((n, hrows, cL), jnp.bfloat16)] * 2
        ospec = pl.BlockSpec((1, rows, cL), lambda b, i: (b, i, 0))
        out_specs += [ospec, ospec]
    if sse:
        out_shapes.append(jax.ShapeDtypeStruct((n * nb, 8, cL), jnp.float32))
        out_specs.append(
            pl.BlockSpec((1, 8, cL), lambda b, i: (b * nb + i, 0, 0)))

    body = functools.partial(_chain_kernel, th=th, wp=wp, wdata=wdata, hb=hb,
                             couts=couts, pool=pool, sse=sse, keep=keep,
                             wpair=wpair, nchw=nchw, nb=nb)
    return pl.pallas_call(
        body,
        out_shape=tuple(out_shapes),
        grid_spec=pltpu.PrefetchScalarGridSpec(
            num_scalar_prefetch=0,
            grid=(n, nb),
            in_specs=[top, main, bot, top, main, bot] + wspecs,
            out_specs=tuple(out_specs),
        ),
        compiler_params=_cparams(),
    )(xP, xP, xP, xG, xG, xG, *wargs)


def _pair_view(x, h, wp):
    n, _, c = x.shape
    return x.reshape(n, h, wp // 2, 2 * c).reshape(n, h * (wp // 2), 2 * c)


def kernel(pred_im, gt,
           conv1_1_w, conv1_1_b, conv1_2_w, conv1_2_b,
           conv2_1_w, conv2_1_b, conv2_2_w, conv2_2_b,
           conv3_1_w, conv3_1_b, conv3_2_w, conv3_2_b,
           conv3_3_w, conv3_3_b):
    n, nc, h, w = pred_im.shape
    wp1 = ((w + 2) + 15) // 16 * 16 + 16
    xP = pred_im.reshape(n, nc, h * w)
    xG = gt.reshape(n, nc, h * w)
    wp2, wp3 = wp1 // 2, wp1 // 4

    yP, yG = _conv_chain(
        xP, xG, [(conv1_1_w, conv1_1_b)],
        h=h, wp=wp1, wdata=w, th=16, hb=1, nchw=True)
    pP, pG, s1 = _conv_chain(
        yP, yG, [(conv1_2_w, conv1_2_b)],
        h=h, wp=wp1, wdata=w, th=16, hb=1, pool=True, sse=True)
    yP, yG = _conv_chain(
        _pair_view(pP, h // 2, wp1), _pair_view(pG, h // 2, wp1),
        [(conv2_1_w, conv2_1_b)],
        h=h // 2, wp=wp2, wdata=w // 2, th=32, hb=1, wpair=True)
    pP, pG, s2 = _conv_chain(
        yP, yG, [(conv2_2_w, conv2_2_b)],
        h=h // 2, wp=wp2, wdata=w // 2, th=32, hb=1, pool=True, sse=True)
    yP, yG = _conv_chain(
        _pair_view(pP, h // 4, wp2), _pair_view(pG, h // 4, wp2),
        [(conv3_1_w, conv3_1_b)],
        h=h // 4, wp=wp3, wdata=w // 4, th=32, hb=1, wpair=True)
    yP, yG = _conv_chain(
        yP, yG, [(conv3_2_w, conv3_2_b)],
        h=h // 4, wp=wp3, wdata=w // 4, th=32, hb=1)
    (s3,) = _conv_chain(
        yP, yG, [(conv3_3_w, conv3_3_b)],
        h=h // 4, wp=wp3, wdata=w // 4, th=32, hb=1,
        sse=True, keep=False)

    n1 = n * h * w * conv1_2_w.shape[0]
    n2 = n * (h // 2) * (w // 2) * conv2_2_w.shape[0]
    n3 = n * (h // 4) * (w // 4) * conv3_3_w.shape[0]
    return (jnp.sum(s1) / n1 + jnp.sum(s2) / n2 + jnp.sum(s3) / n3) / 3.0


# speedup vs baseline: 1.0845x; 1.0420x over previous
import functools

import jax
import jax.numpy as jnp
from jax.experimental import pallas as pl
from jax.experimental.pallas import tpu as pltpu

_VMEM_LIMIT = 48 * 1024 * 1024


def _cparams():
    return pltpu.CompilerParams(
        dimension_semantics=("parallel", "parallel"),
        vmem_limit_bytes=_VMEM_LIMIT,
        internal_scratch_in_bytes=8 * 1024 * 1024)


def _chain_kernel(tP, aP, bP, tG, aG, bG, *rest,
                  th, wp, wdata, hb, couts, pool, sse, keep,
                  wpair, nchw, nb):
    nl = len(couts)
    wrefs = rest[:2 * nl]
    outs = rest[2 * nl:]
    i = pl.program_id(1)
    nr0 = th + 2 * hb

    def build(t_ref, a_ref, bo_ref):
        top = jnp.where(i > 0, t_ref[0], jnp.zeros_like(t_ref[0]))
        bot = jnp.where(i < nb - 1, bo_ref[0], jnp.zeros_like(bo_ref[0]))
        if nchw:
            x3 = jnp.concatenate([top, a_ref[0], bot], axis=1)
            xt = jnp.transpose(x3).astype(jnp.bfloat16)
            x4 = xt.reshape(nr0, wdata, x3.shape[0])
            band = jnp.concatenate(
                [x4, jnp.zeros((nr0, wp - wdata, x3.shape[0]), jnp.bfloat16)],
                axis=1).reshape(nr0 * wp, x3.shape[0])
        else:
            band = jnp.concatenate([top, a_ref[0], bot], axis=0)
            if wpair:
                c = band.shape[-1] // 2
                band = jnp.maximum(band[:, :c], band[:, c:])
        return band

    def stage(bandP, bandG, l):
        nrows = nr0 - 2 * (l + 1)
        m = nrows * wp
        cout = couts[l]

        w_ref = wrefs[2 * l]
        cin = w_ref.shape[0] // 3

        off = hb - (l + 1)
        lo = jnp.where(i > 0, 0, off)
        hi = jnp.where(i < nb - 1, nrows, nrows - off)
        rows = jax.lax.broadcasted_iota(jnp.int32, (nrows, wp, cout), 0)
        cols = jax.lax.broadcasted_iota(jnp.int32, (nrows, wp, cout), 1)
        keepm = (cols < wdata) & (rows >= lo) & (rows < hi)

        def epi(uh):
            o = jnp.maximum(uh, 0.0).astype(jnp.bfloat16).reshape(
                nrows, wp, cout)
            return jnp.where(keepm, o, jnp.zeros_like(o)).reshape(m, cout)

        def combine(z, mh):
            return (pltpu.roll(z[:, :cout], 1, axis=0) + z[:, cout:2 * cout]
                    + pltpu.roll(z[:, 2 * cout:], mh - 1, axis=0)
                    + wrefs[2 * l + 1][...])

        def one(band):
            fold = jnp.concatenate(
                [band[0:m], band[wp:wp + m], band[2 * wp:2 * wp + m]],
                axis=1)
            z = jnp.dot(fold, w_ref[...],
                        preferred_element_type=jnp.float32)
            return epi(combine(z, m))

        return one(bandP), one(bandG)

    def run(t_ref, a_ref, bo_ref):
        return build(t_ref, a_ref, bo_ref)

    bandP = run(tP, aP, bP)
    bandG = run(tG, aG, bG)
    for l in range(nl):
        bandP, bandG = stage(bandP, bandG, l)
    if hb > nl:
        bandP = bandP[(hb - nl) * wp:(hb - nl) * wp + th * wp]
        bandG = bandG[(hb - nl) * wp:(hb - nl) * wp + th * wp]
    oP, oG = bandP, bandG
    cout = couts[-1]
    j = 0
    if keep:
        if pool:
            pPr = oP.reshape(th // 2, 2, wp, cout)
            pGr = oG.reshape(th // 2, 2, wp, cout)
            outs[j][0] = jnp.maximum(pPr[:, 0], pPr[:, 1]).reshape(
                (th // 2) * wp, cout)
            outs[j + 1][0] = jnp.maximum(pGr[:, 0], pGr[:, 1]).reshape(
                (th // 2) * wp, cout)
        else:
            outs[j][0] = oP
            outs[j + 1][0] = oG
        j += 2
    if sse:
        d = oP.astype(jnp.float32) - oG.astype(jnp.float32)
        s1 = jnp.sum((d * d).reshape(th, wp, cout), axis=0)
        outs[j][0] = jnp.sum(s1.reshape(wp // 8, 8, cout), axis=0)


def _conv_chain(xP, xG, wbs, *, h, wp, wdata, th, hb,
                pool=False, sse=False, keep=True, wpair=False, nchw=False):
    n = xP.shape[0]
    couts = [wb[0].shape[0] for wb in wbs]
    th = min(th, h)
    nb = h // th

    if nchw:
        cin0 = xP.shape[1]
        top = pl.BlockSpec((1, cin0, hb * wdata),
                           lambda b, i: (b, 0, jnp.maximum(i * (th // hb) - 1, 0)))
        main = pl.BlockSpec((1, cin0, th * wdata), lambda b, i: (b, 0, i))
        bot = pl.BlockSpec(
            (1, cin0, hb * wdata),
            lambda b, i: (b, 0, jnp.minimum((i + 1) * (th // hb), h // hb - 1)))
    else:
        clanes = xP.shape[2]
        top = pl.BlockSpec((1, hb * wp, clanes),
                           lambda b, i: (b, jnp.maximum(i * (th // hb) - 1, 0), 0))
        main = pl.BlockSpec((1, th * wp, clanes), lambda b, i: (b, i, 0))
        bot = pl.BlockSpec(
            (1, hb * wp, clanes),
            lambda b, i: (b, jnp.minimum((i + 1) * (th // hb), h // hb - 1), 0))

    wargs, wspecs = [], []
    for w_oihw, bias in wbs:
        cin = w_oihw.shape[1]
        cout = w_oihw.shape[0]
        wargs.append(jnp.transpose(w_oihw, (2, 1, 3, 0)).reshape(
            3 * cin, 3 * cout).astype(jnp.bfloat16))
        wspecs.append(pl.BlockSpec((3 * cin, 3 * cout), lambda b, i: (0, 0)))
        wargs.append(bias.reshape(1, cout).astype(jnp.float32))
        wspecs.append(pl.BlockSpec((1, cout), lambda b, i: (0, 0)))

    out_shapes, out_specs = [], []
    cL = couts[-1]
    if keep:
        rows = (th // 2) * wp if pool else th * wp
        hrows = (h // 2) * wp if pool else h * wp
        out_shapes += [jax.ShapeDtypeStruct((n, hrows, cL), jnp.bfloat16)] * 2
        ospec = pl.BlockSpec((1, rows, cL), lambda b, i: (b, i, 0))
        out_specs += [ospec, ospec]
    if sse:
        out_shapes.append(jax.ShapeDtypeStruct((n * nb, 8, cL), jnp.float32))
        out_specs.append(
            pl.BlockSpec((1, 8, cL), lambda b, i: (b * nb + i, 0, 0)))

    body = functools.partial(_chain_kernel, th=th, wp=wp, wdata=wdata, hb=hb,
                             couts=couts, pool=pool, sse=sse, keep=keep,
                             wpair=wpair, nchw=nchw, nb=nb)
    return pl.pallas_call(
        body,
        out_shape=tuple(out_shapes),
        grid_spec=pltpu.PrefetchScalarGridSpec(
            num_scalar_prefetch=0,
            grid=(n, nb),
            in_specs=[top, main, bot, top, main, bot] + wspecs,
            out_specs=tuple(out_specs),
        ),
        compiler_params=_cparams(),
    )(xP, xP, xP, xG, xG, xG, *wargs)


def _pair_view(x, h, wp):
    n, _, c = x.shape
    return x.reshape(n, h, wp // 2, 2 * c).reshape(n, h * (wp // 2), 2 * c)


def kernel(pred_im, gt,
           conv1_1_w, conv1_1_b, conv1_2_w, conv1_2_b,
           conv2_1_w, conv2_1_b, conv2_2_w, conv2_2_b,
           conv3_1_w, conv3_1_b, conv3_2_w, conv3_2_b,
           conv3_3_w, conv3_3_b):
    n, nc, h, w = pred_im.shape
    wp1 = ((w + 2) + 15) // 16 * 16 + 16
    xP = pred_im.reshape(n, nc, h * w)
    xG = gt.reshape(n, nc, h * w)
    wp2, wp3 = wp1 // 2, wp1 // 4

    yP, yG = _conv_chain(
        xP, xG, [(conv1_1_w, conv1_1_b)],
        h=h, wp=wp1, wdata=w, th=16, hb=1, nchw=True)
    pP, pG, s1 = _conv_chain(
        yP, yG, [(conv1_2_w, conv1_2_b)],
        h=h, wp=wp1, wdata=w, th=16, hb=1, pool=True, sse=True)
    yP, yG = _conv_chain(
        _pair_view(pP, h // 2, wp1), _pair_view(pG, h // 2, wp1),
        [(conv2_1_w, conv2_1_b)],
        h=h // 2, wp=wp2, wdata=w // 2, th=32, hb=1, wpair=True)
    pP, pG, s2 = _conv_chain(
        yP, yG, [(conv2_2_w, conv2_2_b)],
        h=h // 2, wp=wp2, wdata=w // 2, th=32, hb=1, pool=True, sse=True)
    yP, yG = _conv_chain(
        _pair_view(pP, h // 4, wp2), _pair_view(pG, h // 4, wp2),
        [(conv3_1_w, conv3_1_b)],
        h=h // 4, wp=wp3, wdata=w // 4, th=32, hb=1, wpair=True)
    yP, yG = _conv_chain(
        yP, yG, [(conv3_2_w, conv3_2_b)],
        h=h // 4, wp=wp3, wdata=w // 4, th=32, hb=1)
    (s3,) = _conv_chain(
        yP, yG, [(conv3_3_w, conv3_3_b)],
        h=h // 4, wp=wp3, wdata=w // 4, th=32, hb=1,
        sse=True, keep=False)

    n1 = n * h * w * conv1_2_w.shape[0]
    n2 = n * (h // 2) * (w // 2) * conv2_2_w.shape[0]
    n3 = n * (h // 4) * (w // 4) * conv3_3_w.shape[0]
    return (jnp.sum(s1) / n1 + jnp.sum(s2) / n2 + jnp.sum(s3) / n3) / 3.0


# speedup vs baseline: 1.1184x; 1.0313x over previous
import functools

import jax
import jax.numpy as jnp
from jax.experimental import pallas as pl
from jax.experimental.pallas import tpu as pltpu

_VMEM_LIMIT = 56 * 1024 * 1024


def _cparams():
    return pltpu.CompilerParams(
        dimension_semantics=("parallel", "parallel"),
        vmem_limit_bytes=_VMEM_LIMIT,
        internal_scratch_in_bytes=8 * 1024 * 1024)


def _chain_kernel(tP, aP, bP, tG, aG, bG, *rest,
                  th, wp, wdata, hb, couts, pool, sse, keep,
                  wpair, nchw, nb):
    nl = len(couts)
    wrefs = rest[:2 * nl]
    outs = rest[2 * nl:]
    i = pl.program_id(1)
    nr0 = th + 2 * hb

    def build(t_ref, a_ref, bo_ref):
        top = jnp.where(i > 0, t_ref[0], jnp.zeros_like(t_ref[0]))
        bot = jnp.where(i < nb - 1, bo_ref[0], jnp.zeros_like(bo_ref[0]))
        if nchw:
            x3 = jnp.concatenate([top, a_ref[0], bot], axis=1)
            xt = jnp.transpose(x3).astype(jnp.bfloat16)
            x4 = xt.reshape(nr0, wdata, x3.shape[0])
            band = jnp.concatenate(
                [x4, jnp.zeros((nr0, wp - wdata, x3.shape[0]), jnp.bfloat16)],
                axis=1).reshape(nr0 * wp, x3.shape[0])
        else:
            band = jnp.concatenate([top, a_ref[0], bot], axis=0)
            if wpair:
                c = band.shape[-1] // 2
                band = jnp.maximum(band[:, :c], band[:, c:])
        return band

    def stage(bandP, bandG, l):
        nrows = nr0 - 2 * (l + 1)
        m = nrows * wp
        cout = couts[l]

        w_ref = wrefs[2 * l]
        cin = w_ref.shape[0] // 3

        off = hb - (l + 1)
        lo = jnp.where(i > 0, 0, off)
        hi = jnp.where(i < nb - 1, nrows, nrows - off)
        rows = jax.lax.broadcasted_iota(jnp.int32, (nrows, wp, cout), 0)
        cols = jax.lax.broadcasted_iota(jnp.int32, (nrows, wp, cout), 1)
        keepm = (cols < wdata) & (rows >= lo) & (rows < hi)

        def epi(uh):
            o = jnp.maximum(uh, 0.0).astype(jnp.bfloat16).reshape(
                nrows, wp, cout)
            return jnp.where(keepm, o, jnp.zeros_like(o)).reshape(m, cout)

        def combine(z, mh):
            return (pltpu.roll(z[:, :cout], 1, axis=0) + z[:, cout:2 * cout]
                    + pltpu.roll(z[:, 2 * cout:], mh - 1, axis=0)
                    + wrefs[2 * l + 1][...])

        def one(band):
            fold = jnp.concatenate(
                [band[0:m], band[wp:wp + m], band[2 * wp:2 * wp + m]],
                axis=1)
            z = jnp.dot(fold, w_ref[...],
                        preferred_element_type=jnp.float32)
            return epi(combine(z, m))

        return one(bandP), one(bandG)

    def run(t_ref, a_ref, bo_ref):
        return build(t_ref, a_ref, bo_ref)

    bandP = run(tP, aP, bP)
    bandG = run(tG, aG, bG)
    for l in range(nl):
        bandP, bandG = stage(bandP, bandG, l)
    if hb > nl:
        bandP = bandP[(hb - nl) * wp:(hb - nl) * wp + th * wp]
        bandG = bandG[(hb - nl) * wp:(hb - nl) * wp + th * wp]
    oP, oG = bandP, bandG
    cout = couts[-1]
    j = 0
    if keep:
        if pool:
            pPr = oP.reshape(th // 2, 2, wp, cout)
            pGr = oG.reshape(th // 2, 2, wp, cout)
            outs[j][0] = jnp.maximum(pPr[:, 0], pPr[:, 1]).reshape(
                (th // 2) * wp, cout)
            outs[j + 1][0] = jnp.maximum(pGr[:, 0], pGr[:, 1]).reshape(
                (th // 2) * wp, cout)
        else:
            outs[j][0] = oP
            outs[j + 1][0] = oG
        j += 2
    if sse:
        d = oP.astype(jnp.float32) - oG.astype(jnp.float32)
        s1 = jnp.sum((d * d).reshape(th, wp, cout), axis=0)
        outs[j][0] = jnp.sum(s1.reshape(wp // 8, 8, cout), axis=0)


def _conv_chain(xP, xG, wbs, *, h, wp, wdata, th, hb,
                pool=False, sse=False, keep=True, wpair=False, nchw=False):
    n = xP.shape[0]
    couts = [wb[0].shape[0] for wb in wbs]
    th = min(th, h)
    nb = h // th

    if nchw:
        cin0 = xP.shape[1]
        top = pl.BlockSpec((1, cin0, hb * wdata),
                           lambda b, i: (b, 0, jnp.maximum(i * (th // hb) - 1, 0)))
        main = pl.BlockSpec((1, cin0, th * wdata), lambda b, i: (b, 0, i))
        bot = pl.BlockSpec(
            (1, cin0, hb * wdata),
            lambda b, i: (b, 0, jnp.minimum((i + 1) * (th // hb), h // hb - 1)))
    else:
        clanes = xP.shape[2]
        top = pl.BlockSpec((1, hb * wp, clanes),
                           lambda b, i: (b, jnp.maximum(i * (th // hb) - 1, 0), 0))
        main = pl.BlockSpec((1, th * wp, clanes), lambda b, i: (b, i, 0))
        bot = pl.BlockSpec(
            (1, hb * wp, clanes),
            lambda b, i: (b, jnp.minimum((i + 1) * (th // hb), h // hb - 1), 0))

    wargs, wspecs = [], []
    for w_oihw, bias in wbs:
        cin = w_oihw.shape[1]
        cout = w_oihw.shape[0]
        wargs.append(jnp.transpose(w_oihw, (2, 1, 3, 0)).reshape(
            3 * cin, 3 * cout).astype(jnp.bfloat16))
        wspecs.append(pl.BlockSpec((3 * cin, 3 * cout), lambda b, i: (0, 0)))
        wargs.append(bias.reshape(1, cout).astype(jnp.float32))
        wspecs.append(pl.BlockSpec((1, cout), lambda b, i: (0, 0)))

    out_shapes, out_specs = [], []
    cL = couts[-1]
    if keep:
        rows = (th // 2) * wp if pool else th * wp
        hrows = (h // 2) * wp if pool else h * wp
        out_shapes += [jax.ShapeDtypeStruct((n, hrows, cL), jnp.bfloat16)] * 2
        ospec = pl.BlockSpec((1, rows, cL), lambda b, i: (b, i, 0))
        out_specs += [ospec, ospec]
    if sse:
        out_shapes.append(jax.ShapeDtypeStruct((n * nb, 8, cL), jnp.float32))
        out_specs.append(
            pl.BlockSpec((1, 8, cL), lambda b, i: (b * nb + i, 0, 0)))

    body = functools.partial(_chain_kernel, th=th, wp=wp, wdata=wdata, hb=hb,
                             couts=couts, pool=pool, sse=sse, keep=keep,
                             wpair=wpair, nchw=nchw, nb=nb)
    return pl.pallas_call(
        body,
        out_shape=tuple(out_shapes),
        grid_spec=pltpu.PrefetchScalarGridSpec(
            num_scalar_prefetch=0,
            grid=(n, nb),
            in_specs=[top, main, bot, top, main, bot] + wspecs,
            out_specs=tuple(out_specs),
        ),
        compiler_params=_cparams(),
    )(xP, xP, xP, xG, xG, xG, *wargs)


def _pair_view(x, h, wp):
    n, _, c = x.shape
    return x.reshape(n, h, wp // 2, 2 * c).reshape(n, h * (wp // 2), 2 * c)


def kernel(pred_im, gt,
           conv1_1_w, conv1_1_b, conv1_2_w, conv1_2_b,
           conv2_1_w, conv2_1_b, conv2_2_w, conv2_2_b,
           conv3_1_w, conv3_1_b, conv3_2_w, conv3_2_b,
           conv3_3_w, conv3_3_b):
    n, nc, h, w = pred_im.shape
    wp1 = ((w + 2) + 15) // 16 * 16 + 16
    xP = pred_im.reshape(n, nc, h * w)
    xG = gt.reshape(n, nc, h * w)
    wp2, wp3 = wp1 // 2, wp1 // 4

    yP, yG = _conv_chain(
        xP, xG, [(conv1_1_w, conv1_1_b)],
        h=h, wp=wp1, wdata=w, th=64, hb=1, nchw=True)
    pP, pG, s1 = _conv_chain(
        yP, yG, [(conv1_2_w, conv1_2_b)],
        h=h, wp=wp1, wdata=w, th=64, hb=1, pool=True, sse=True)
    yP, yG = _conv_chain(
        _pair_view(pP, h // 2, wp1), _pair_view(pG, h // 2, wp1),
        [(conv2_1_w, conv2_1_b)],
        h=h // 2, wp=wp2, wdata=w // 2, th=64, hb=1, wpair=True)
    pP, pG, s2 = _conv_chain(
        yP, yG, [(conv2_2_w, conv2_2_b)],
        h=h // 2, wp=wp2, wdata=w // 2, th=64, hb=1, pool=True, sse=True)
    yP, yG = _conv_chain(
        _pair_view(pP, h // 4, wp2), _pair_view(pG, h // 4, wp2),
        [(conv3_1_w, conv3_1_b)],
        h=h // 4, wp=wp3, wdata=w // 4, th=64, hb=1, wpair=True)
    yP, yG = _conv_chain(
        yP, yG, [(conv3_2_w, conv3_2_b)],
        h=h // 4, wp=wp3, wdata=w // 4, th=64, hb=1)
    (s3,) = _conv_chain(
        yP, yG, [(conv3_3_w, conv3_3_b)],
        h=h // 4, wp=wp3, wdata=w // 4, th=64, hb=1,
        sse=True, keep=False)

    n1 = n * h * w * conv1_2_w.shape[0]
    n2 = n * (h // 2) * (w // 2) * conv2_2_w.shape[0]
    n3 = n * (h // 4) * (w // 4) * conv3_3_w.shape[0]
    return (jnp.sum(s1) / n1 + jnp.sum(s2) / n2 + jnp.sum(s3) / n3) / 3.0
